# baseline (speedup 1.0000x reference)
import jax
import jax.numpy as jnp
from jax.experimental import pallas as pl
from jax.experimental.pallas import tpu as pltpu

N = 4096
BR = 128
NB = N // BR

_LAYER_DIMS = [(256, 256), (256, 128), (128, 86), (86, 64), (64, 32),
               (32, 16), (16, 8), (8, 16)]


RB = 4096


def _fused_gcn_kernel(adj_ref, dis_ref, x_ref, *wb_refs_and_out):
    w_refs = wb_refs_and_out[0:16:2]
    b_refs = wb_refs_and_out[1:16:2]
    out_ref = wb_refs_and_out[16]
    a_scr = wb_refs_and_out[17]
    s_bufs = (wb_refs_and_out[18], wb_refs_and_out[19])

    i = pl.program_id(0)

    @pl.when(i == 0)
    def _s1():
        S1 = jnp.dot(x_ref[...], w_refs[0][...],
                     preferred_element_type=jnp.float32)
        s_bufs[0][:, :256] = S1.astype(jnp.bfloat16)

    a_scr[pl.ds(i * BR, BR), :] = (adj_ref[...] * dis_ref[...]).astype(jnp.bfloat16)

    o1 = jnp.dot(a_scr[pl.ds(i * BR, BR), :], s_bufs[0][:, :256],
                 preferred_element_type=jnp.float32) + b_refs[0][...]
    s_bufs[1][pl.ds(i * BR, BR), :128] = jnp.dot(
        jnp.maximum(o1, 0.0), w_refs[1][...],
        preferred_element_type=jnp.float32).astype(jnp.bfloat16)

    @pl.when(i == NB - 1)
    def _compute():
        for l in range(1, 8):
            dout = _LAYER_DIMS[l][1]
            s_cur = s_bufs[l % 2]
            b = b_refs[l][...]

            def body(rb, _, l=l, dout=dout, b=b, s_cur=s_cur):
                a_blk = a_scr[pl.ds(rb * RB, RB), :]
                o = jnp.dot(a_blk, s_cur[:, :dout],
                            preferred_element_type=jnp.float32) + b
                if l < 7:
                    dnxt = _LAYER_DIMS[l + 1][1]
                    s_nxt = jnp.dot(jnp.maximum(o, 0.0), w_refs[l + 1][...],
                                    preferred_element_type=jnp.float32)
                    s_bufs[(l + 1) % 2][pl.ds(rb * RB, RB), :dnxt] = (
                        s_nxt.astype(jnp.bfloat16))
                else:
                    out_ref[pl.ds(rb * RB, RB), :] = o
                return 0

            jax.lax.fori_loop(0, N // RB, body, 0)


def kernel(x, adj, dis, W1, b1, W2, b2, W3, b3, W4, b4, W5, b5, W6, b6,
           W7, b7, W8, b8):
    ws = [W1, W2, W3, W4, W5, W6, W7, W8]
    bs = [b1, b2, b3, b4, b5, b6, b7, b8]

    wb_specs = []
    wb_args = []
    for w, b in zip(ws, bs):
        wb_specs.append(pl.BlockSpec(w.shape, lambda i: (0, 0)))
        wb_args.append(w)
        wb_specs.append(pl.BlockSpec((1, b.shape[0]), lambda i: (0, 0)))
        wb_args.append(b.reshape(1, -1))

    dout = _LAYER_DIMS[-1][1]
    out = pl.pallas_call(
        _fused_gcn_kernel,
        grid=(NB,),
        in_specs=[
            pl.BlockSpec((BR, N), lambda i: (i, 0)),
            pl.BlockSpec((BR, N), lambda i: (i, 0)),
            pl.BlockSpec((N, 256), lambda i: (0, 0)),
        ] + wb_specs,
        out_specs=pl.BlockSpec((N, dout), lambda i: (0, 0)),
        out_shape=jax.ShapeDtypeStruct((N, dout), jnp.float32),
        scratch_shapes=[
            pltpu.VMEM((N, N), jnp.bfloat16),
            pltpu.VMEM((N, 256), jnp.bfloat16),
            pltpu.VMEM((N, 256), jnp.bfloat16),
        ],
        compiler_params=pltpu.CompilerParams(
            dimension_semantics=("arbitrary",),
            vmem_limit_bytes=63 * 1024 * 1024,
        ),
    )(adj, dis, x, *wb_args)
    return out


# speedup vs baseline: 1.9095x; 1.9095x over previous
import jax
import jax.numpy as jnp
from jax.experimental import pallas as pl
from jax.experimental.pallas import tpu as pltpu

N = 4096
BR = 256
NB = N // BR

_LAYER_DIMS = [(256, 256), (256, 128), (128, 86), (86, 64), (64, 32),
               (32, 16), (16, 8), (8, 16)]


RB = 2048


def _fused_gcn_kernel(adj_ref, dis_ref, x_ref, *wb_refs_and_out):
    w_refs = wb_refs_and_out[0:16:2]
    b_refs = wb_refs_and_out[1:16:2]
    out_ref = wb_refs_and_out[16]
    a_scr = wb_refs_and_out[17]
    s_bufs = (wb_refs_and_out[18], wb_refs_and_out[19])

    i = pl.program_id(0)

    @pl.when(i == 0)
    def _s1():
        S1 = jnp.dot(x_ref[...], w_refs[0][...],
                     preferred_element_type=jnp.float32)
        s_bufs[0][:, :256] = S1.astype(jnp.bfloat16)

    a_scr[pl.ds(i * BR, BR), :] = (adj_ref[...] * dis_ref[...]).astype(jnp.bfloat16)

    o1 = jnp.dot(a_scr[pl.ds(i * BR, BR), :], s_bufs[0][:, :256],
                 preferred_element_type=jnp.float32) + b_refs[0][...]
    s_bufs[1][pl.ds(i * BR, BR), :128] = jnp.dot(
        jnp.maximum(o1, 0.0), w_refs[1][...],
        preferred_element_type=jnp.float32).astype(jnp.bfloat16)

    @pl.when(i == NB - 1)
    def _compute():
        for l in range(1, 8):
            dout = _LAYER_DIMS[l][1]
            s_cur = s_bufs[l % 2]
            b = b_refs[l][...]

            def body(rb, _, l=l, dout=dout, b=b, s_cur=s_cur):
                a_blk = a_scr[pl.ds(rb * RB, RB), :]
                o = jnp.dot(a_blk, s_cur[:, :dout],
                            preferred_element_type=jnp.float32) + b
                if l < 7:
                    dnxt = _LAYER_DIMS[l + 1][1]
                    s_nxt = jnp.dot(jnp.maximum(o, 0.0), w_refs[l + 1][...],
                                    preferred_element_type=jnp.float32)
                    s_bufs[(l + 1) % 2][pl.ds(rb * RB, RB), :dnxt] = (
                        s_nxt.astype(jnp.bfloat16))
                else:
                    out_ref[pl.ds(rb * RB, RB), :] = o
                return 0

            jax.lax.fori_loop(0, N // RB, body, 0)


def kernel(x, adj, dis, W1, b1, W2, b2, W3, b3, W4, b4, W5, b5, W6, b6,
           W7, b7, W8, b8):
    ws = [W1, W2, W3, W4, W5, W6, W7, W8]
    bs = [b1, b2, b3, b4, b5, b6, b7, b8]

    wb_specs = []
    wb_args = []
    for w, b in zip(ws, bs):
        wb_specs.append(pl.BlockSpec(w.shape, lambda i: (0, 0)))
        wb_args.append(w)
        wb_specs.append(pl.BlockSpec((1, b.shape[0]), lambda i: (0, 0)))
        wb_args.append(b.reshape(1, -1))

    dout = _LAYER_DIMS[-1][1]
    out = pl.pallas_call(
        _fused_gcn_kernel,
        grid=(NB,),
        in_specs=[
            pl.BlockSpec((BR, N), lambda i: (i, 0)),
            pl.BlockSpec((BR, N), lambda i: (i, 0)),
            pl.BlockSpec((N, 256), lambda i: (0, 0)),
        ] + wb_specs,
        out_specs=pl.BlockSpec((N, dout), lambda i: (0, 0)),
        out_shape=jax.ShapeDtypeStruct((N, dout), jnp.float32),
        scratch_shapes=[
            pltpu.VMEM((N, N), jnp.bfloat16),
            pltpu.VMEM((N, 256), jnp.bfloat16),
            pltpu.VMEM((N, 256), jnp.bfloat16),
        ],
        compiler_params=pltpu.CompilerParams(
            dimension_semantics=("arbitrary",),
            vmem_limit_bytes=67043328,
        ),
    )(adj, dis, x, *wb_args)
    return out
